# Initial kernel scaffold; baseline (speedup 1.0000x reference)
import jax
import jax.numpy as jnp
from jax.experimental import pallas as pl


def kernel(agent_ids, table):
    raise NotImplementedError("write your pallas kernel here")


# speedup vs baseline: 1.4356x; 1.4356x over previous
import functools

import jax
import jax.numpy as jnp
from jax import lax
from jax.experimental import pallas as pl
from jax.experimental.pallas import tpu as pltpu
from jax.experimental.pallas import tpu_sc as plsc

_BATCH = 16384
_FIELDS = 26
_EMBED = 32
_TOTAL = _BATCH * _FIELDS
_NC = 2
_NS = 16
_NW = _NC * _NS
_PER_W = _TOTAL // _NW
_CHUNK = 128
_N_CHUNKS = _PER_W // _CHUNK


def _sc_gather(idx2d, table):
    mesh = plsc.VectorSubcoreMesh(core_axis_name="c", subcore_axis_name="s")

    @functools.partial(
        pl.kernel,
        mesh=mesh,
        out_type=jax.ShapeDtypeStruct((_TOTAL, _EMBED), jnp.float32),
        scratch_types=[
            pltpu.VMEM((_N_CHUNKS, _CHUNK), jnp.int32),
            pltpu.VMEM((_CHUNK, _EMBED), jnp.float32),
            pltpu.SemaphoreType.DMA,
        ],
        compiler_params=pltpu.CompilerParams(use_tc_tiling_on_sc=False),
    )
    def k(idx_hbm, table_hbm, out_hbm, idx_v, rows_v, sem):
        wid = lax.axis_index("s") * _NC + lax.axis_index("c")
        crow = wid * _N_CHUNKS
        pltpu.sync_copy(idx_hbm.at[pl.ds(crow, _N_CHUNKS)], idx_v)

        def body(j, carry):
            pltpu.async_copy(table_hbm.at[idx_v.at[j]], rows_v, sem).wait()
            pltpu.sync_copy(
                rows_v, out_hbm.at[pl.ds((crow + j) * _CHUNK, _CHUNK)])
            return carry

        lax.fori_loop(0, _N_CHUNKS, body, 0)

    return k(idx2d, table)


def kernel(agent_ids, table):
    idx2d = agent_ids.reshape(_TOTAL // _CHUNK, _CHUNK)
    out = _sc_gather(idx2d, table)
    return out.reshape(_BATCH, _FIELDS, _EMBED)


# speedup vs baseline: 1.5533x; 1.0820x over previous
import functools

import jax
import jax.numpy as jnp
from jax import lax
from jax.experimental import pallas as pl
from jax.experimental.pallas import tpu as pltpu
from jax.experimental.pallas import tpu_sc as plsc

_BATCH = 16384
_FIELDS = 26
_EMBED = 32
_TOTAL = _BATCH * _FIELDS
_NC = 2
_NS = 16
_NW = _NC * _NS
_PER_W = _TOTAL // _NW
_CHUNK = 128
_N_CHUNKS = _PER_W // _CHUNK
_G = 4
_GROUP_ROWS = _G * _CHUNK
_NG = _N_CHUNKS // _G


def _sc_gather(idx2d, table):
    mesh = plsc.VectorSubcoreMesh(core_axis_name="c", subcore_axis_name="s")

    @functools.partial(
        pl.kernel,
        mesh=mesh,
        out_type=jax.ShapeDtypeStruct((_TOTAL, _EMBED), jnp.float32),
        scratch_types=[
            pltpu.VMEM((_N_CHUNKS, _CHUNK), jnp.int32),
            pltpu.VMEM((_GROUP_ROWS, _EMBED), jnp.float32),
            pltpu.VMEM((_GROUP_ROWS, _EMBED), jnp.float32),
            pltpu.SemaphoreType.DMA,
            pltpu.SemaphoreType.DMA,
            pltpu.SemaphoreType.DMA,
            pltpu.SemaphoreType.DMA,
        ],
        compiler_params=pltpu.CompilerParams(use_tc_tiling_on_sc=False),
    )
    def k(idx_hbm, table_hbm, out_hbm, idx_v,
          buf0, buf1, gsem0, gsem1, osem0, osem1):
        wid = lax.axis_index("s") * _NC + lax.axis_index("c")
        crow = wid * _N_CHUNKS
        row0 = crow * _CHUNK
        pltpu.sync_copy(idx_hbm.at[pl.ds(crow, _N_CHUNKS)], idx_v)

        bufs = (buf0, buf1)
        gsems = (gsem0, gsem1)
        osems = (osem0, osem1)

        def run_group(g, half):
            buf, gsem, osem = bufs[half], gsems[half], osems[half]

            @pl.when(g >= 2)
            def _():
                pltpu.make_async_copy(
                    buf, out_hbm.at[pl.ds(0, _GROUP_ROWS)], osem).wait()

            for i in range(_G):
                pltpu.async_copy(
                    table_hbm.at[idx_v.at[g * _G + i]],
                    buf.at[pl.ds(i * _CHUNK, _CHUNK)], gsem)
            for i in range(_G):
                pltpu.make_async_copy(
                    table_hbm.at[idx_v.at[0]],
                    buf.at[pl.ds(i * _CHUNK, _CHUNK)], gsem).wait()
            pltpu.async_copy(
                buf, out_hbm.at[pl.ds(row0 + g * _GROUP_ROWS, _GROUP_ROWS)],
                osem)

        def body(t, carry):
            run_group(2 * t, 0)
            run_group(2 * t + 1, 1)
            return carry

        lax.fori_loop(0, _NG // 2, body, 0)
        pltpu.make_async_copy(
            buf0, out_hbm.at[pl.ds(0, _GROUP_ROWS)], osem0).wait()
        pltpu.make_async_copy(
            buf1, out_hbm.at[pl.ds(0, _GROUP_ROWS)], osem1).wait()

    return k(idx2d, table)


def kernel(agent_ids, table):
    idx2d = agent_ids.reshape(_TOTAL // _CHUNK, _CHUNK)
    out = _sc_gather(idx2d, table)
    return out.reshape(_BATCH, _FIELDS, _EMBED)
